# Initial kernel scaffold; baseline (speedup 1.0000x reference)
import jax
import jax.numpy as jnp
from jax.experimental import pallas as pl


def kernel(tokens, item_mem):
    raise NotImplementedError("write your pallas kernel here")


# speedup vs baseline: 19.8443x; 19.8443x over previous
import functools

import jax
import jax.numpy as jnp
from jax import lax
from jax.experimental import pallas as pl
from jax.experimental.pallas import tpu as pltpu
from jax.experimental.pallas import tpu_sc as plsc

B, L, VOCAB, D = 1024, 20, 1000, 2048
PAD = L - 1
WEXT = 2080
NC, NS = 2, 16
NW = NC * NS
BPW = B // NW


def _sc_encode(tokens, t_ext):
    mesh = plsc.VectorSubcoreMesh(core_axis_name="c", subcore_axis_name="s")

    @functools.partial(
        pl.kernel,
        out_type=jax.ShapeDtypeStruct((B, D), jnp.int32),
        mesh=mesh,
        compiler_params=pltpu.CompilerParams(
            use_tc_tiling_on_sc=False, needs_layout_passes=False),
        scratch_types=[
            pltpu.VMEM((BPW, L), jnp.int32),
            pltpu.VMEM((L, WEXT), jnp.float32),
            pltpu.VMEM((L, WEXT), jnp.float32),
            pltpu.VMEM((D,), jnp.int32),
            pltpu.VMEM((D,), jnp.int32),
            pltpu.SemaphoreType.DMA,
            pltpu.SemaphoreType.DMA,
            pltpu.SemaphoreType.DMA,
            pltpu.SemaphoreType.DMA,
        ],
    )
    def enc(tok_hbm, text_hbm, out_hbm, tok_v, rows0, rows1, out0, out1,
            sem0, sem1, osem0, osem1):
        wid = lax.axis_index("s") * NC + lax.axis_index("c")
        base = wid * BPW
        pltpu.sync_copy(tok_hbm.at[pl.ds(base, BPW)], tok_v)

        rbufs, rsems = (rows0, rows1), (sem0, sem1)
        obufs, osems = (out0, out1), (osem0, osem1)
        gathers = [None, None]
        scatters = [None, None]

        gathers[0] = pltpu.async_copy(text_hbm.at[tok_v.at[0]], rows0, sem0)
        for b in range(BPW):
            p = b % 2
            if b + 1 < BPW:
                gathers[1 - p] = pltpu.async_copy(
                    text_hbm.at[tok_v.at[b + 1]], rbufs[1 - p], rsems[1 - p])
            gathers[p].wait()
            rows = rbufs[p]
            if scatters[p] is not None:
                scatters[p].wait()
            ov = obufs[p]

            ci = lax.iota(jnp.int32, 16)
            rsplat = [jnp.full((16,), l, jnp.int32) for l in range(L)]

            def jbody(jc, _, rows=rows, ov=ov):
                col = jc * 16
                acc = plsc.load_gather(rows, [rsplat[0], ci + (col + PAD)])
                for l in range(1, L):
                    acc = acc + plsc.load_gather(
                        rows, [rsplat[l], ci + (col + PAD - l)])
                ov[pl.ds(col, 16)] = jnp.where(
                    acc > 0.0, jnp.int32(1), jnp.int32(-1))
                return 0

            lax.fori_loop(0, D // 16, jbody, 0)
            scatters[p] = pltpu.async_copy(ov, out_hbm.at[base + b], osems[p])
        for s in scatters:
            if s is not None:
                s.wait()

    return enc(tokens, t_ext)


def kernel(tokens, item_mem):
    t_ext = jnp.concatenate(
        [item_mem[:, D - PAD:], item_mem, item_mem[:, :WEXT - D - PAD]],
        axis=1)
    return _sc_encode(tokens, t_ext)


# speedup vs baseline: 20.3234x; 1.0241x over previous
import functools

import jax
import jax.numpy as jnp
from jax import lax
from jax.experimental import pallas as pl
from jax.experimental.pallas import tpu as pltpu
from jax.experimental.pallas import tpu_sc as plsc

B, L, VOCAB, D = 1024, 20, 1000, 2048
NC, NS = 2, 16
NW = NC * NS
BPW = B // NW


def _sc_encode(tokens, item_mem):
    mesh = plsc.VectorSubcoreMesh(core_axis_name="c", subcore_axis_name="s")

    @functools.partial(
        pl.kernel,
        out_type=jax.ShapeDtypeStruct((B, D), jnp.int32),
        mesh=mesh,
        compiler_params=pltpu.CompilerParams(
            use_tc_tiling_on_sc=False, needs_layout_passes=False),
        scratch_types=[
            pltpu.VMEM((BPW, L), jnp.int32),
            pltpu.VMEM((L, D), jnp.float32),
            pltpu.VMEM((L, D), jnp.float32),
            pltpu.VMEM((D,), jnp.int32),
            pltpu.VMEM((D,), jnp.int32),
            pltpu.SemaphoreType.DMA,
            pltpu.SemaphoreType.DMA,
            pltpu.SemaphoreType.DMA,
            pltpu.SemaphoreType.DMA,
        ],
    )
    def enc(tok_hbm, tab_hbm, out_hbm, tok_v, rows0, rows1, out0, out1,
            sem0, sem1, osem0, osem1):
        wid = lax.axis_index("s") * NC + lax.axis_index("c")
        base = wid * BPW
        pltpu.sync_copy(tok_hbm.at[pl.ds(base, BPW)], tok_v)

        rbufs, rsems = (rows0, rows1), (sem0, sem1)
        obufs, osems = (out0, out1), (osem0, osem1)
        gathers = [None, None]
        scatters = [None, None]

        ci = lax.iota(jnp.int32, 16)
        rsplat = [jnp.full((16,), l, jnp.int32) for l in range(L)]
        cil = [ci - l for l in range(L)]

        gathers[0] = pltpu.async_copy(tab_hbm.at[tok_v.at[0]], rows0, sem0)
        for b in range(BPW):
            p = b % 2
            if b + 1 < BPW:
                gathers[1 - p] = pltpu.async_copy(
                    tab_hbm.at[tok_v.at[b + 1]], rbufs[1 - p], rsems[1 - p])
            gathers[p].wait()
            rows = rbufs[p]
            if scatters[p] is not None:
                scatters[p].wait()
            ov = obufs[p]

            def jbody(jc, _, rows=rows, ov=ov, wrap=False):
                col = jc * 16
                acc = plsc.load_gather(rows, [rsplat[0], cil[0] + col])
                for l in range(1, L):
                    idx = cil[l] + col
                    if wrap:
                        idx = idx & (D - 1)
                    acc = acc + plsc.load_gather(rows, [rsplat[l], idx])
                ov[pl.ds(col, 16)] = jnp.where(
                    acc > 0.0, jnp.int32(1), jnp.int32(-1))
                return 0

            lax.fori_loop(0, D // 16, functools.partial(jbody, wrap=True), 0)
            scatters[p] = pltpu.async_copy(ov, out_hbm.at[base + b], osems[p])
        for s in scatters:
            if s is not None:
                s.wait()

    return enc(tokens, item_mem)


def kernel(tokens, item_mem):
    return _sc_encode(tokens, item_mem)


# speedup vs baseline: 30.8110x; 1.5160x over previous
import functools

import jax
import jax.numpy as jnp
from jax import lax
from jax.experimental import pallas as pl
from jax.experimental.pallas import tpu as pltpu
from jax.experimental.pallas import tpu_sc as plsc

B, L, VOCAB, D = 1024, 20, 1000, 2048
W = D // 2
HALO = 32
WH = W + HALO
NC, NS = 2, 16
NW = NC * NS
BPW = B // NW
NCHUNK = W // 16


def _sc_encode(tokens, tab):
    mesh = plsc.VectorSubcoreMesh(core_axis_name="c", subcore_axis_name="s")

    @functools.partial(
        pl.kernel,
        out_type=jax.ShapeDtypeStruct((B, D), jnp.int32),
        mesh=mesh,
        compiler_params=pltpu.CompilerParams(
            use_tc_tiling_on_sc=False, needs_layout_passes=False),
        scratch_types=[
            pltpu.VMEM((BPW, L), jnp.int32),
            pltpu.VMEM((L, WH), jnp.int32),
            pltpu.VMEM((L, WH), jnp.int32),
            pltpu.VMEM((D,), jnp.int32),
            pltpu.VMEM((D,), jnp.int32),
            pltpu.SemaphoreType.DMA,
            pltpu.SemaphoreType.DMA,
            pltpu.SemaphoreType.DMA,
            pltpu.SemaphoreType.DMA,
        ],
    )
    def enc(tok_hbm, tab_hbm, out_hbm, tok_v, rows0, rows1, out0, out1,
            sem0, sem1, osem0, osem1):
        wid = lax.axis_index("s") * NC + lax.axis_index("c")
        base = wid * BPW
        pltpu.sync_copy(tok_hbm.at[pl.ds(base, BPW)], tok_v)

        rbufs, rsems = (rows0, rows1), (sem0, sem1)
        obufs, osems = (out0, out1), (osem0, osem1)
        gathers = [None, None]
        scatters = [None, None]

        ci = lax.iota(jnp.int32, 16)
        rsplat = [jnp.full((16,), l, jnp.int32) for l in range(L)]
        cil = [ci + (HALO - l) for l in range(L)]
        one = jnp.full((16,), 1, jnp.int32)
        mone = jnp.full((16,), -1, jnp.int32)

        gathers[0] = pltpu.async_copy(tab_hbm.at[tok_v.at[0]], rows0, sem0)
        for b in range(BPW):
            p = b % 2
            if b + 1 < BPW:
                gathers[1 - p] = pltpu.async_copy(
                    tab_hbm.at[tok_v.at[b + 1]], rbufs[1 - p], rsems[1 - p])
            gathers[p].wait()
            rows = rbufs[p]
            if scatters[p] is not None:
                scatters[p].wait()
            ov = obufs[p]

            def jbody(jc, _, rows=rows, ov=ov):
                col = jc * 16
                acc = plsc.bitcast(
                    plsc.load_gather(rows, [rsplat[0], cil[0] + col]),
                    jnp.bfloat16)
                for l in range(1, L):
                    acc = acc + plsc.bitcast(
                        plsc.load_gather(rows, [rsplat[l], cil[l] + col]),
                        jnp.bfloat16)
                lo, hi = plsc.unpack(acc, format=plsc.PackFormat.INTERLEAVED)
                ov[pl.ds(col, 16)] = jnp.where(lo > 0.0, one, mone)
                ov[pl.ds(col + W, 16)] = jnp.where(hi > 0.0, one, mone)
                return 0

            lax.fori_loop(0, NCHUNK, jbody, 0)
            scatters[p] = pltpu.async_copy(ov, out_hbm.at[base + b], osems[p])
        for s in scatters:
            if s is not None:
                s.wait()

    return enc(tokens, tab)


def kernel(tokens, item_mem):
    imb = item_mem.astype(jnp.bfloat16)
    lo = jnp.concatenate([imb[:, D - HALO:], imb[:, :W]], axis=1)
    hi = jnp.concatenate([imb[:, W - HALO:W], imb[:, W:]], axis=1)
    tab = lax.bitcast_convert_type(
        jnp.stack([lo, hi], axis=-1), jnp.int32)
    return _sc_encode(tokens, tab)


# speedup vs baseline: 36.5766x; 1.1871x over previous
import functools

import jax
import jax.numpy as jnp
from jax import lax
from jax.experimental import pallas as pl
from jax.experimental.pallas import tpu as pltpu
from jax.experimental.pallas import tpu_sc as plsc

B, L, VOCAB, D = 1024, 20, 1000, 2048
W = D // 2
HALO = 32
WH = W + HALO
NC, NS = 2, 16
NW = NC * NS
BPW = B // NW
NCHUNK = W // 16


def _sc_encode(tokens, tab):
    mesh = plsc.VectorSubcoreMesh(core_axis_name="c", subcore_axis_name="s")

    @functools.partial(
        pl.kernel,
        out_type=jax.ShapeDtypeStruct((B, D), jnp.int32),
        mesh=mesh,
        compiler_params=pltpu.CompilerParams(
            use_tc_tiling_on_sc=False, needs_layout_passes=False),
        scratch_types=[
            pltpu.VMEM((BPW, L), jnp.int32),
            pltpu.VMEM((L, WH), jnp.int32),
            pltpu.VMEM((L, WH), jnp.int32),
            pltpu.VMEM((D,), jnp.int32),
            pltpu.VMEM((D,), jnp.int32),
            pltpu.SemaphoreType.DMA,
            pltpu.SemaphoreType.DMA,
            pltpu.SemaphoreType.DMA,
            pltpu.SemaphoreType.DMA,
        ],
    )
    def enc(tok_hbm, tab_hbm, out_hbm, tok_v, rows0, rows1, out0, out1,
            sem0, sem1, osem0, osem1):
        wid = lax.axis_index("s") * NC + lax.axis_index("c")
        base = wid * BPW
        pltpu.sync_copy(tok_hbm.at[pl.ds(base, BPW)], tok_v)

        rbufs, rsems = (rows0, rows1), (sem0, sem1)
        obufs, osems = (out0, out1), (osem0, osem1)
        gathers = [None, None]
        scatters = [None, None]

        ci = lax.iota(jnp.int32, 16)
        rsplat = [jnp.full((16,), l, jnp.int32) for l in range(L)]
        cil = [ci + (HALO - l) for l in range(L)]
        one = jnp.full((16,), 1, jnp.int32)
        mone = jnp.full((16,), -1, jnp.int32)

        gathers[0] = pltpu.async_copy(tab_hbm.at[tok_v.at[0]], rows0, sem0)
        for b in range(BPW):
            p = b % 2
            if b + 1 < BPW:
                gathers[1 - p] = pltpu.async_copy(
                    tab_hbm.at[tok_v.at[b + 1]], rbufs[1 - p], rsems[1 - p])
            gathers[p].wait()
            rows = rbufs[p]
            if scatters[p] is not None:
                scatters[p].wait()
            ov = obufs[p]

            def chunk(col, rows, ov):
                terms = [
                    plsc.bitcast(
                        plsc.load_gather(rows, [rsplat[l], cil[l] + col]),
                        jnp.bfloat16)
                    for l in range(L)
                ]
                while len(terms) > 1:
                    terms = [a + b for a, b in zip(terms[::2], terms[1::2])] \
                        + ([terms[-1]] if len(terms) % 2 else [])
                lo, hi = plsc.unpack(
                    terms[0], format=plsc.PackFormat.INTERLEAVED)
                ov[pl.ds(col, 16)] = jnp.where(lo > 0.0, one, mone)
                ov[pl.ds(col + W, 16)] = jnp.where(hi > 0.0, one, mone)

            def jbody(jc, _, rows=rows, ov=ov):
                col = jc * 32
                chunk(col, rows, ov)
                chunk(col + 16, rows, ov)
                return 0

            lax.fori_loop(0, NCHUNK // 2, jbody, 0)
            scatters[p] = pltpu.async_copy(ov, out_hbm.at[base + b], osems[p])
        for s in scatters:
            if s is not None:
                s.wait()

    return enc(tokens, tab)


def kernel(tokens, item_mem):
    s = lax.bitcast_convert_type(item_mem, jnp.uint32)
    slo = jnp.concatenate([s[:, D - HALO:], s[:, :W]], axis=1)
    shi = jnp.concatenate([s[:, W - HALO:W], s[:, W:]], axis=1)
    word = (jnp.uint32(0x3F803F80)
            | ((slo >> 16) & jnp.uint32(0x8000))
            | (shi & jnp.uint32(0x80000000)))
    tab = lax.bitcast_convert_type(word, jnp.int32)
    return _sc_encode(tokens, tab)
